# baseline (device time: 26279 ns/iter reference)
import jax
import jax.numpy as jnp
from jax import lax
from jax.experimental import pallas as pl
from jax.experimental.pallas import tpu as pltpu

N_DEV = 4
SUB = 4
_GELU_C = 0.7978845608028654


def _gelu(y):
    return 0.5 * y * (1.0 + jnp.tanh(_GELU_C * (y + 0.044715 * y * y * y)))


def kernel(x, w_mat):
    m_per, k = x.shape
    _, n_per = w_mat.shape
    k2 = k // 2
    m_sub = m_per // SUB

    def body(x_ref, w_ref, out_ref, comm_r, comm_l,
             send_r, recv_r, send_l, recv_l):
        my = lax.axis_index("i")
        left = lax.rem(my + N_DEV - 1, N_DEV)
        right = lax.rem(my + 1, N_DEV)

        barrier_sem = pltpu.get_barrier_semaphore()
        for nbr in (left, right):
            pl.semaphore_signal(
                barrier_sem, inc=1,
                device_id=(nbr,), device_id_type=pl.DeviceIdType.MESH,
            )
        pl.semaphore_wait(barrier_sem, 2)

        def row(origin):
            return pl.ds(lax.rem(origin + N_DEV, N_DEV) * m_per, m_per)

        def dot(a, b):
            return jnp.dot(a, b, preferred_element_type=jnp.float32)

        def mk(ring, d, s):
            comm, ssem, rsem, dev, col0 = (
                (comm_r, send_r, recv_r, right, 0) if ring == 0
                else (comm_l, send_l, recv_l, left, k2))
            rows = pl.ds(s * m_sub, m_sub)
            if d == 1:
                src = x_ref.at[rows, pl.ds(col0, k2)]
            else:
                src = comm.at[d - 1, rows, :]
            return pltpu.make_async_remote_copy(
                src_ref=src,
                dst_ref=comm.at[d, rows, :],
                send_sem=ssem.at[d, s],
                recv_sem=rsem.at[d, s],
                device_id=(dev,), device_id_type=pl.DeviceIdType.MESH,
            )

        descs = {}

        for s in range(SUB):
            for ring in (0, 1):
                r = mk(ring, 1, s)
                r.start()
                descs[(ring, 1, s)] = r

        out_ref[row(my), :] = _gelu(dot(x_ref[...], w_ref[...]))

        w_top = w_ref[:k2, :]
        w_bot = w_ref[k2:, :]

        for d in (2, 3):
            for s in range(SUB):
                for ring in (0, 1):
                    descs[(ring, d - 1, s)].wait_recv()
                    r = mk(ring, d, s)
                    r.start()
                    descs[(ring, d, s)] = r
            if d == 2:
                out_ref[row(my - 1), :] = dot(comm_r[1], w_top)
                out_ref[row(my + 1), :] = dot(comm_l[1], w_bot)
            else:
                out_ref[row(my + 2), :] = _gelu(
                    dot(comm_r[2], w_top) + dot(comm_l[2], w_bot))

        for s in range(SUB):
            descs[(0, 3, s)].wait_recv()
            descs[(1, 3, s)].wait_recv()
        out_ref[row(my - 1), :] = _gelu(
            out_ref[row(my - 1), :] + dot(comm_l[3], w_bot))
        out_ref[row(my + 1), :] = _gelu(
            out_ref[row(my + 1), :] + dot(comm_r[3], w_top))

        for r in descs.values():
            r.wait_send()

    return pl.pallas_call(
        body,
        out_shape=jax.ShapeDtypeStruct((N_DEV * m_per, n_per), jnp.float32),
        in_specs=[
            pl.BlockSpec(memory_space=pltpu.VMEM),
            pl.BlockSpec(memory_space=pltpu.VMEM),
        ],
        out_specs=pl.BlockSpec(memory_space=pltpu.VMEM),
        scratch_shapes=[
            pltpu.VMEM((N_DEV, m_per, k2), jnp.float32),
            pltpu.VMEM((N_DEV, m_per, k2), jnp.float32),
            pltpu.SemaphoreType.DMA((N_DEV, SUB)),
            pltpu.SemaphoreType.DMA((N_DEV, SUB)),
            pltpu.SemaphoreType.DMA((N_DEV, SUB)),
            pltpu.SemaphoreType.DMA((N_DEV, SUB)),
        ],
        compiler_params=pltpu.CompilerParams(collective_id=0),
    )(x, w_mat)


# device time: 6572 ns/iter; 3.9986x vs baseline; 3.9986x over previous
import jax
import jax.numpy as jnp
from jax import lax
from jax.experimental import pallas as pl
from jax.experimental.pallas import tpu as pltpu

N_DEV = 4
SUB = 2
_GELU_C = 0.7978845608028654


def _gelu(y):
    return 0.5 * y * (1.0 + jnp.tanh(_GELU_C * (y + 0.044715 * y * y * y)))


def kernel(x, w_mat):
    m_per, k = x.shape
    _, n_per = w_mat.shape
    k2 = k // 2
    m_sub = m_per // SUB

    def body(x_ref, w_ref, out_ref, comm_r, comm_l,
             send_r, recv_r, send_l, recv_l):
        my = lax.axis_index("i")
        left = lax.rem(my + N_DEV - 1, N_DEV)
        right = lax.rem(my + 1, N_DEV)

        barrier_sem = pltpu.get_barrier_semaphore()
        for nbr in (left, right):
            pl.semaphore_signal(
                barrier_sem, inc=1,
                device_id=(nbr,), device_id_type=pl.DeviceIdType.MESH,
            )
        pl.semaphore_wait(barrier_sem, 2)

        def row(origin):
            return pl.ds(lax.rem(origin + N_DEV, N_DEV) * m_per, m_per)

        def dot(a, b):
            return jnp.dot(a, b, preferred_element_type=jnp.float32)

        def mk(ring, d, s):
            comm, ssem, rsem, dev, col0 = (
                (comm_r, send_r, recv_r, right, 0) if ring == 0
                else (comm_l, send_l, recv_l, left, k2))
            rows = pl.ds(s * m_sub, m_sub)
            if d == 1:
                src = x_ref.at[rows, pl.ds(col0, k2)]
            else:
                src = comm.at[d - 1, rows, :]
            return pltpu.make_async_remote_copy(
                src_ref=src,
                dst_ref=comm.at[d, rows, :],
                send_sem=ssem.at[d, s],
                recv_sem=rsem.at[d, s],
                device_id=(dev,), device_id_type=pl.DeviceIdType.MESH,
            )

        descs = {}

        for s in range(SUB):
            for ring in (0, 1):
                r = mk(ring, 1, s)
                r.start()
                descs[(ring, 1, s)] = r

        out_ref[row(my), :] = _gelu(dot(x_ref[...], w_ref[...]))

        w_top = w_ref[:k2, :]
        w_bot = w_ref[k2:, :]

        for d in (2, 3):
            for s in range(SUB):
                for ring in (0, 1):
                    descs[(ring, d - 1, s)].wait_recv()
                    r = mk(ring, d, s)
                    r.start()
                    descs[(ring, d, s)] = r
            if d == 2:
                out_ref[row(my - 1), :] = dot(comm_r[1], w_top)
                out_ref[row(my + 1), :] = dot(comm_l[1], w_bot)
            else:
                out_ref[row(my + 2), :] = _gelu(
                    dot(comm_r[2], w_top) + dot(comm_l[2], w_bot))

        for s in range(SUB):
            descs[(0, 3, s)].wait_recv()
            descs[(1, 3, s)].wait_recv()
        out_ref[row(my - 1), :] = _gelu(
            out_ref[row(my - 1), :] + dot(comm_l[3], w_bot))
        out_ref[row(my + 1), :] = _gelu(
            out_ref[row(my + 1), :] + dot(comm_r[3], w_top))

        for r in descs.values():
            r.wait_send()

    return pl.pallas_call(
        body,
        out_shape=jax.ShapeDtypeStruct((N_DEV * m_per, n_per), jnp.float32),
        in_specs=[
            pl.BlockSpec(memory_space=pltpu.VMEM),
            pl.BlockSpec(memory_space=pltpu.VMEM),
        ],
        out_specs=pl.BlockSpec(memory_space=pltpu.VMEM),
        scratch_shapes=[
            pltpu.VMEM((N_DEV, m_per, k2), jnp.float32),
            pltpu.VMEM((N_DEV, m_per, k2), jnp.float32),
            pltpu.SemaphoreType.DMA((N_DEV, SUB)),
            pltpu.SemaphoreType.DMA((N_DEV, SUB)),
            pltpu.SemaphoreType.DMA((N_DEV, SUB)),
            pltpu.SemaphoreType.DMA((N_DEV, SUB)),
        ],
        compiler_params=pltpu.CompilerParams(collective_id=0),
    )(x, w_mat)
